# baseline (speedup 1.0000x reference)
import functools

import jax
import jax.numpy as jnp
from jax import lax
from jax.experimental import pallas as pl
from jax.experimental.pallas import tpu as pltpu
from jax.experimental.pallas import tpu_sc as plsc

NC = 2
NS = 16
L = 16
D = 16
B = 16384
NW = NC * NS
PW = B // NW
K = 4
NB = PW // K
PARS = 7
NV = 1000000


def _gmf_body(uidx_hbm, iidx_hbm, ut_hbm, it_hbm, wvec_hbm, bvec_hbm,
              out_hbm, u_sm, i_sm, bufs, out_v, wv_v, bv_v, sems):
    c = lax.axis_index("c")
    s = lax.axis_index("s")
    wid = s * NC + c
    base = wid * PW

    pltpu.sync_copy(wvec_hbm, wv_v)
    pltpu.sync_copy(bvec_hbm, bv_v)
    wv = wv_v[...]
    bv = bv_v[...]
    lanes = lax.iota(jnp.int32, L)

    pltpu.sync_copy(uidx_hbm.at[pl.ds(base, PW)], u_sm)
    pltpu.sync_copy(iidx_hbm.at[pl.ds(base, PW)], i_sm)

    def fire(uvec, ivec, tg, par):
        for e in range(K):
            u = uvec[tg * K + e]
            u128 = pl.multiple_of((u >> 7) << 7, 128)
            pltpu.async_copy(ut_hbm.at[:, pl.ds(u128, 128)],
                             bufs.at[par, 0, e], sems.at[par])
            i = ivec[tg * K + e]
            i128 = pl.multiple_of((i >> 7) << 7, 128)
            pltpu.async_copy(it_hbm.at[:, pl.ds(i128, 128)],
                             bufs.at[par, 1, e], sems.at[par])

    def drain(par):
        for _ in range(2 * K):
            pltpu.make_async_copy(ut_hbm.at[:, pl.ds(0, 128)],
                                  bufs.at[par, 0, 0], sems.at[par]).wait()

    def process(uvec, ivec, t, par, acc):
        for e in range(K):
            u = uvec[t * K + e]
            i = ivec[t * K + e]
            cu = jnp.full((L,), u & 127, dtype=jnp.int32)
            ci = jnp.full((L,), i & 127, dtype=jnp.int32)
            ev = jnp.full((L,), e, dtype=jnp.int32)
            zv = jnp.zeros((L,), jnp.int32)
            ov = jnp.ones((L,), jnp.int32)
            ug = plsc.load_gather(bufs, [zv + par, zv, ev, lanes, cu])
            ig = plsc.load_gather(bufs, [zv + par, ov, ev, lanes, ci])
            dot = jnp.sum(ug * ig)
            acc = jnp.where(lanes == t * K + e, dot, acc)
        return acc

    NG = PW // L
    uv0 = u_sm[pl.ds(0, L)]
    iv0 = i_sm[pl.ds(0, L)]
    uv1 = u_sm[pl.ds(L, L)]
    iv1 = i_sm[pl.ds(L, L)]
    for t in range(4):
        fire(uv0, iv0, t, t)
    fire(uv1, iv1, 0, 4)
    fire(uv1, iv1, 1, 5)
    fire(uv1, iv1, 2, 6)

    def body(k, carry):
        uv = u_sm[pl.ds(k * L, L)]
        iv = i_sm[pl.ds(k * L, L)]
        n1 = jnp.minimum(k + 1, NG - 1) * L
        un1 = u_sm[pl.ds(n1, L)]
        in1 = i_sm[pl.ds(n1, L)]
        n2 = jnp.minimum(k + 2, NG - 1) * L
        un2 = u_sm[pl.ds(n2, L)]
        in2 = i_sm[pl.ds(n2, L)]
        acc = jnp.zeros((L,), jnp.float32)
        for t in range(4):
            par = (k * 4 + t) % PARS
            drain(par)
            acc = process(uv, iv, t, par, acc)
            if t == 0:
                fire(un1, in1, 3, par)
            else:
                fire(un2, in2, t - 1, par)
        logits = acc * wv + bv
        out_v[pl.ds(k * L, L)] = 1.0 / (1.0 + jnp.exp(-logits))
        return carry

    lax.fori_loop(0, NG, body, 0)
    for t in range(PARS):
        drain(t)

    pltpu.sync_copy(out_v, out_hbm.at[pl.ds(base, PW)])


@jax.jit
def _gmf(uidx, iidx, ut_t, it_t, wvec, bvec):
    mesh = plsc.VectorSubcoreMesh(
        core_axis_name="c", subcore_axis_name="s",
        num_cores=NC, num_subcores=NS)
    run = functools.partial(
        pl.kernel,
        out_type=jax.ShapeDtypeStruct((B,), jnp.float32),
        mesh=mesh,
        compiler_params=pltpu.CompilerParams(
            needs_layout_passes=False, use_tc_tiling_on_sc=True),
        scratch_types=[
            pltpu.VMEM((PW,), jnp.int32),
            pltpu.VMEM((PW,), jnp.int32),
            pltpu.VMEM((PARS, 2, K, D, 128), jnp.float32),
            pltpu.VMEM((PW,), jnp.float32),
            pltpu.VMEM((L,), jnp.float32),
            pltpu.VMEM((L,), jnp.float32),
            pltpu.SemaphoreType.DMA((PARS,)),
        ],
    )(_gmf_body)
    return run(uidx, iidx, ut_t, it_t, wvec, bvec)


def kernel(user_indices, item_indices, user_table, item_table, dense_w, dense_b):
    uidx = user_indices.astype(jnp.int32)
    iidx = item_indices.astype(jnp.int32)
    wvec = jnp.full((L,), dense_w[0, 0], dtype=jnp.float32)
    bvec = jnp.full((L,), dense_b[0], dtype=jnp.float32)
    out = _gmf(uidx, iidx, user_table.T, item_table.T, wvec, bvec)
    return out.reshape(B, 1)


# speedup vs baseline: 1.0055x; 1.0055x over previous
import functools

import jax
import jax.numpy as jnp
from jax import lax
from jax.experimental import pallas as pl
from jax.experimental.pallas import tpu as pltpu
from jax.experimental.pallas import tpu_sc as plsc

NC = 2
NS = 16
L = 16
D = 16
B = 16384
NW = NC * NS
PW = B // NW
K = 4
NB = PW // K
PARS = 6
NV = 1000000


def _gmf_body(uidx_hbm, iidx_hbm, ut_hbm, it_hbm, wvec_hbm, bvec_hbm,
              out_hbm, u_sm, i_sm, bufs, out_v, wv_v, bv_v, sems):
    c = lax.axis_index("c")
    s = lax.axis_index("s")
    wid = s * NC + c
    base = wid * PW

    pltpu.sync_copy(wvec_hbm, wv_v)
    pltpu.sync_copy(bvec_hbm, bv_v)
    wv = wv_v[...]
    bv = bv_v[...]
    lanes = lax.iota(jnp.int32, L)

    pltpu.sync_copy(uidx_hbm.at[pl.ds(base, PW)], u_sm)
    pltpu.sync_copy(iidx_hbm.at[pl.ds(base, PW)], i_sm)

    def fire(uvec, ivec, tg, par):
        for e in range(K):
            u = uvec[tg * K + e]
            u128 = pl.multiple_of((u >> 7) << 7, 128)
            pltpu.async_copy(ut_hbm.at[:, pl.ds(u128, 128)],
                             bufs.at[par, 0, e], sems.at[par])
            i = ivec[tg * K + e]
            i128 = pl.multiple_of((i >> 7) << 7, 128)
            pltpu.async_copy(it_hbm.at[:, pl.ds(i128, 128)],
                             bufs.at[par, 1, e], sems.at[par])

    def drain(par):
        for _ in range(2 * K):
            pltpu.make_async_copy(ut_hbm.at[:, pl.ds(0, 128)],
                                  bufs.at[par, 0, 0], sems.at[par]).wait()

    def process(uvec, ivec, t, par, acc):
        for e in range(K):
            u = uvec[t * K + e]
            i = ivec[t * K + e]
            cu = jnp.full((L,), u & 127, dtype=jnp.int32)
            ci = jnp.full((L,), i & 127, dtype=jnp.int32)
            ev = jnp.full((L,), e, dtype=jnp.int32)
            zv = jnp.zeros((L,), jnp.int32)
            ov = jnp.ones((L,), jnp.int32)
            ug = plsc.load_gather(bufs, [zv + par, zv, ev, lanes, cu])
            ig = plsc.load_gather(bufs, [zv + par, ov, ev, lanes, ci])
            dot = jnp.sum(ug * ig)
            acc = jnp.where(lanes == t * K + e, dot, acc)
        return acc

    NG = PW // L
    uv0 = u_sm[pl.ds(0, L)]
    iv0 = i_sm[pl.ds(0, L)]
    uv1 = u_sm[pl.ds(L, L)]
    iv1 = i_sm[pl.ds(L, L)]
    for t in range(4):
        fire(uv0, iv0, t, t)
    fire(uv1, iv1, 0, 4)
    fire(uv1, iv1, 1, 5)

    def body(k, carry):
        uv = u_sm[pl.ds(k * L, L)]
        iv = i_sm[pl.ds(k * L, L)]
        n1 = jnp.minimum(k + 1, NG - 1) * L
        un1 = u_sm[pl.ds(n1, L)]
        in1 = i_sm[pl.ds(n1, L)]
        n2 = jnp.minimum(k + 2, NG - 1) * L
        un2 = u_sm[pl.ds(n2, L)]
        in2 = i_sm[pl.ds(n2, L)]
        acc = jnp.zeros((L,), jnp.float32)
        for t in range(4):
            par = (k * 4 + t) % PARS
            drain(par)
            acc = process(uv, iv, t, par, acc)
            if t < 2:
                fire(un1, in1, t + 2, par)
            else:
                fire(un2, in2, t - 2, par)
        logits = acc * wv + bv
        out_v[pl.ds(k * L, L)] = 1.0 / (1.0 + jnp.exp(-logits))
        return carry

    lax.fori_loop(0, NG, body, 0)
    for t in range(PARS):
        drain(t)

    pltpu.sync_copy(out_v, out_hbm.at[pl.ds(base, PW)])


@jax.jit
def _gmf(uidx, iidx, ut_t, it_t, wvec, bvec):
    mesh = plsc.VectorSubcoreMesh(
        core_axis_name="c", subcore_axis_name="s",
        num_cores=NC, num_subcores=NS)
    run = functools.partial(
        pl.kernel,
        out_type=jax.ShapeDtypeStruct((B,), jnp.float32),
        mesh=mesh,
        compiler_params=pltpu.CompilerParams(
            needs_layout_passes=False, use_tc_tiling_on_sc=True),
        scratch_types=[
            pltpu.VMEM((PW,), jnp.int32),
            pltpu.VMEM((PW,), jnp.int32),
            pltpu.VMEM((PARS, 2, K, D, 128), jnp.float32),
            pltpu.VMEM((PW,), jnp.float32),
            pltpu.VMEM((L,), jnp.float32),
            pltpu.VMEM((L,), jnp.float32),
            pltpu.SemaphoreType.DMA((PARS,)),
        ],
    )(_gmf_body)
    return run(uidx, iidx, ut_t, it_t, wvec, bvec)


def kernel(user_indices, item_indices, user_table, item_table, dense_w, dense_b):
    uidx = user_indices.astype(jnp.int32)
    iidx = item_indices.astype(jnp.int32)
    wvec = jnp.full((L,), dense_w[0, 0], dtype=jnp.float32)
    bvec = jnp.full((L,), dense_b[0], dtype=jnp.float32)
    out = _gmf(uidx, iidx, user_table.T, item_table.T, wvec, bvec)
    return out.reshape(B, 1)
